# Initial kernel scaffold; baseline (speedup 1.0000x reference)
import jax
import jax.numpy as jnp
from jax.experimental import pallas as pl


def kernel(input_tensor, gamma, beta, emb_table):
    raise NotImplementedError("write your pallas kernel here")


# speedup vs baseline: 2.5824x; 2.5824x over previous
import functools

import jax
import jax.numpy as jnp
from jax import lax
from jax.experimental import pallas as pl
from jax.experimental.pallas import tpu as pltpu
from jax.experimental.pallas import tpu_sc as plsc

D_EMB = 16
EMBEDDING_SIZE = 1000
BUFFER = 5
SCALE = EMBEDDING_SIZE / (2 * BUFFER)
NAN_PADDING = EMBEDDING_SIZE
BN_EPS = 1e-5

N_TOTAL = 16384 * 100
LANES = 128
ROWS = N_TOTAL // LANES
TC_CHUNKS = 8
ROWS_PER_CHUNK = ROWS // TC_CHUNKS

NW = 32
ROWS_PER_W = ROWS // NW
CHUNK_ROWS = 16
CHUNK_ELEMS = CHUNK_ROWS * LANES
N_CHUNKS = ROWS_PER_W // CHUNK_ROWS


def _tc_idx_body(x_ref, g_ref, b_ref, idx_ref, s_ref):
    p = pl.program_id(0)
    c = pl.program_id(1)

    @pl.when((p == 0) & (c == 0))
    def _init():
        s_ref[0] = 0.0
        s_ref[1] = 0.0
        s_ref[2] = 0.0

    @pl.when(p == 0)
    def _acc_sum():
        x = x_ref[...]
        nan = x != x
        xv = jnp.where(nan, 0.0, x)
        s_ref[0] += jnp.sum(xv)
        s_ref[2] += jnp.sum(jnp.where(nan, 0.0, 1.0))

    @pl.when((p == 0) & (c == TC_CHUNKS - 1))
    def _fin_mean():
        s_ref[3] = s_ref[0] / s_ref[2]

    @pl.when(p == 1)
    def _acc_var():
        x = x_ref[...]
        nan = x != x
        d = (x - s_ref[3]) ** 2
        s_ref[1] += jnp.sum(jnp.where(nan, 0.0, d))

    @pl.when((p == 1) & (c == TC_CHUNKS - 1))
    def _fin_var():
        var = s_ref[1] / s_ref[2]
        s_ref[4] = jnp.sqrt(var + BN_EPS)

    @pl.when(p == 2)
    def _emit_idx():
        x = x_ref[...]
        normalized = (x - s_ref[3]) / s_ref[4] * g_ref[0] + b_ref[0]
        t = (normalized + BUFFER) * SCALE
        t = jnp.clip(t, 0.0, float(NAN_PADDING - 1))
        ii = t.astype(jnp.int32)
        idx_ref[...] = jnp.where(x != x, NAN_PADDING, ii)


def _compute_idx(x2d, gamma, beta):
    return pl.pallas_call(
        _tc_idx_body,
        grid=(3, TC_CHUNKS),
        in_specs=[
            pl.BlockSpec((ROWS_PER_CHUNK, LANES), lambda p, c: (c, 0)),
            pl.BlockSpec(memory_space=pltpu.SMEM),
            pl.BlockSpec(memory_space=pltpu.SMEM),
        ],
        out_specs=pl.BlockSpec((ROWS_PER_CHUNK, LANES), lambda p, c: (c, 0)),
        out_shape=jax.ShapeDtypeStruct((ROWS, LANES), jnp.int32),
        scratch_shapes=[pltpu.SMEM((8,), jnp.float32)],
    )(x2d, gamma, beta)


_SC_MESH = plsc.VectorSubcoreMesh(core_axis_name="c", subcore_axis_name="s")


@functools.partial(
    pl.kernel,
    mesh=_SC_MESH,
    compiler_params=pltpu.CompilerParams(use_tc_tiling_on_sc=False),
    out_type=jax.ShapeDtypeStruct((N_TOTAL, D_EMB), jnp.float32),
    scratch_types=[
        pltpu.VMEM((CHUNK_ROWS, LANES), jnp.int32),
        pltpu.VMEM((CHUNK_ELEMS, D_EMB), jnp.float32),
        pltpu.SemaphoreType.DMA,
    ],
)
def _sc_gather(idx_hbm, table_hbm, out_hbm, ibuf, obuf, sem):
    wid = lax.axis_index("s") * 2 + lax.axis_index("c")
    row0 = wid * ROWS_PER_W
    base = wid * ROWS_PER_W * LANES

    def chunk_body(c, carry):
        pltpu.sync_copy(idx_hbm.at[pl.ds(row0 + c * CHUNK_ROWS, CHUNK_ROWS)], ibuf)
        handles = [
            pltpu.async_copy(
                table_hbm.at[ibuf.at[j]],
                obuf.at[pl.ds(j * LANES, LANES)],
                sem,
            )
            for j in range(CHUNK_ROWS)
        ]
        for h in handles:
            h.wait()
        pltpu.sync_copy(obuf, out_hbm.at[pl.ds(base + c * CHUNK_ELEMS, CHUNK_ELEMS)])
        return carry

    lax.fori_loop(0, N_CHUNKS, chunk_body, 0)


def kernel(input_tensor, gamma, beta, emb_table):
    shape = input_tensor.shape
    x2d = input_tensor.reshape(ROWS, LANES)
    idx = _compute_idx(x2d, gamma, beta)
    out = _sc_gather(idx, emb_table)
    return out.reshape(shape + (D_EMB,))


# speedup vs baseline: 7.0010x; 2.7110x over previous
import functools

import jax
import jax.numpy as jnp
from jax import lax
from jax.experimental import pallas as pl
from jax.experimental.pallas import tpu as pltpu
from jax.experimental.pallas import tpu_sc as plsc

D_EMB = 16
EMBEDDING_SIZE = 1000
BUFFER = 5
SCALE = EMBEDDING_SIZE / (2 * BUFFER)
NAN_PADDING = EMBEDDING_SIZE
BN_EPS = 1e-5

B_ROWS = 16384
B_COLS = 100
TC_CHUNKS = 8
TC_BLOCK_ROWS = B_ROWS // TC_CHUNKS

NW = 32
ROWS_PER_W = B_ROWS // NW
CHUNK_ROWS = 16
N_CHUNKS = ROWS_PER_W // CHUNK_ROWS


def _tc_idx_body(x_ref, g_ref, b_ref, idx_ref, s_ref):
    p = pl.program_id(0)
    c = pl.program_id(1)

    @pl.when((p == 0) & (c == 0))
    def _init():
        s_ref[0] = 0.0
        s_ref[1] = 0.0
        s_ref[2] = 0.0

    @pl.when(p == 0)
    def _acc_sum():
        x = x_ref[...]
        nan = x != x
        xv = jnp.where(nan, 0.0, x)
        s_ref[0] += jnp.sum(xv)
        s_ref[2] += jnp.sum(jnp.where(nan, 0.0, 1.0))

    @pl.when((p == 0) & (c == TC_CHUNKS - 1))
    def _fin_mean():
        s_ref[3] = s_ref[0] / s_ref[2]

    @pl.when(p == 1)
    def _acc_var():
        x = x_ref[...]
        nan = x != x
        d = (x - s_ref[3]) ** 2
        s_ref[1] += jnp.sum(jnp.where(nan, 0.0, d))

    @pl.when((p == 1) & (c == TC_CHUNKS - 1))
    def _fin_var():
        var = s_ref[1] / s_ref[2]
        s_ref[4] = jnp.sqrt(var + BN_EPS)

    @pl.when(p == 2)
    def _emit_idx():
        x = x_ref[...]
        normalized = (x - s_ref[3]) / s_ref[4] * g_ref[0] + b_ref[0]
        t = (normalized + BUFFER) * SCALE
        t = jnp.clip(t, 0.0, float(NAN_PADDING - 1))
        ii = t.astype(jnp.int32)
        idx_ref[...] = jnp.where(x != x, NAN_PADDING, ii)


def _compute_idx(x, gamma, beta):
    return pl.pallas_call(
        _tc_idx_body,
        grid=(3, TC_CHUNKS),
        in_specs=[
            pl.BlockSpec((TC_BLOCK_ROWS, B_COLS), lambda p, c: (c, 0)),
            pl.BlockSpec(memory_space=pltpu.SMEM),
            pl.BlockSpec(memory_space=pltpu.SMEM),
        ],
        out_specs=pl.BlockSpec((TC_BLOCK_ROWS, B_COLS), lambda p, c: (c, 0)),
        out_shape=jax.ShapeDtypeStruct((B_ROWS, B_COLS), jnp.int32),
        scratch_shapes=[pltpu.SMEM((8,), jnp.float32)],
    )(x, gamma, beta)


_SC_MESH = plsc.VectorSubcoreMesh(core_axis_name="c", subcore_axis_name="s")


@functools.partial(
    pl.kernel,
    mesh=_SC_MESH,
    compiler_params=pltpu.CompilerParams(use_tc_tiling_on_sc=False),
    out_type=jax.ShapeDtypeStruct((B_ROWS, B_COLS, D_EMB), jnp.float32),
    scratch_types=[
        pltpu.VMEM((CHUNK_ROWS, B_COLS), jnp.int32),
        pltpu.VMEM((CHUNK_ROWS, B_COLS), jnp.int32),
        pltpu.VMEM((CHUNK_ROWS, B_COLS, D_EMB), jnp.float32),
        pltpu.VMEM((CHUNK_ROWS, B_COLS, D_EMB), jnp.float32),
        pltpu.SemaphoreType.DMA,
        pltpu.SemaphoreType.DMA,
        pltpu.SemaphoreType.DMA,
    ],
)
def _sc_gather(idx_hbm, table_hbm, out_hbm, ibuf0, ibuf1, obuf0, obuf1,
               sem_g, sem_w0, sem_w1):
    wid = lax.axis_index("s") * 2 + lax.axis_index("c")
    row0 = wid * ROWS_PER_W
    ibufs = (ibuf0, ibuf1)
    obufs = (obuf0, obuf1)
    sems_w = (sem_w0, sem_w1)

    def load_idx(c, b):
        pltpu.sync_copy(idx_hbm.at[pl.ds(row0 + c * CHUNK_ROWS, CHUNK_ROWS)],
                        ibufs[b])

    def gather_chunk(b):
        handles = [
            pltpu.async_copy(table_hbm.at[ibufs[b].at[j]],
                             obufs[b].at[j], sem_g)
            for j in range(CHUNK_ROWS)
        ]
        for h in handles:
            h.wait()

    def start_writeback(c, b):
        pltpu.async_copy(
            obufs[b],
            out_hbm.at[pl.ds(row0 + c * CHUNK_ROWS, CHUNK_ROWS)],
            sems_w[b],
        )

    def wait_writeback(b):
        pltpu.make_async_copy(
            out_hbm.at[pl.ds(row0, CHUNK_ROWS)], obufs[b], sems_w[b]
        ).wait()

    load_idx(0, 0)
    load_idx(1, 1)
    gather_chunk(0)
    start_writeback(0, 0)
    load_idx(2, 0)
    gather_chunk(1)
    start_writeback(1, 1)
    load_idx(3, 1)

    def pair_body(k, carry):
        for b in range(2):
            c = 2 * k + b
            wait_writeback(b)
            gather_chunk(b)
            start_writeback(c, b)
            load_idx(c + 2, b)
        return carry

    lax.fori_loop(1, N_CHUNKS // 2 - 1, pair_body, 0)

    for b in range(2):
        c = N_CHUNKS - 2 + b
        wait_writeback(b)
        gather_chunk(b)
        start_writeback(c, b)
    wait_writeback(0)
    wait_writeback(1)


def kernel(input_tensor, gamma, beta, emb_table):
    idx = _compute_idx(input_tensor, gamma, beta)
    return _sc_gather(idx, emb_table)
